# baseline (speedup 1.0000x reference)
import functools

import jax
import jax.numpy as jnp
from jax import lax
from jax.experimental import pallas as pl
from jax.experimental.pallas import tpu as pltpu
from jax.experimental.pallas import tpu_sc as plsc

_ND = 10000
_NE = 10000
_E = 160000
_H = 128
_L = 3
_TILES = 16
_CHUNK = 128
_NCHUNK = 80
_EPAD = _TILES * _NCHUNK * _CHUNK
_RPAD = 10240
_RPT = _RPAD // _TILES

_MESH = plsc.VectorSubcoreMesh(core_axis_name="c", subcore_axis_name="s")


@functools.partial(
    pl.kernel,
    out_type=jax.ShapeDtypeStruct((2, _RPAD, _H), jnp.float32),
    mesh=_MESH,
    scratch_types=[
        pltpu.VMEM_SHARED((_RPAD, _H), jnp.float32),
        pltpu.VMEM((_NCHUNK, _CHUNK), jnp.int32),
        pltpu.VMEM((_CHUNK,), jnp.int32),
        pltpu.VMEM((_CHUNK,), jnp.int32),
        pltpu.VMEM((_CHUNK, _H), jnp.float32),
        pltpu.VMEM((_CHUNK, _H), jnp.float32),
        pltpu.SemaphoreType.DMA,
        pltpu.SemaphoreType.DMA,
        pltpu.SemaphoreType.DMA,
        pltpu.SemaphoreType.DMA,
        pltpu.SemaphoreType.DMA,
        pltpu.SemaphoreType.DMA,
    ],
)
def _sc_spmm(u_hbm, src_hbm, dst_hbm, out_hbm, acc_sh, srcv,
             dstb0, dstb1, buf0, buf1, gs0, gs1, ss0, ss1, is0, is1):
    c = lax.axis_index("c")
    s = lax.axis_index("s")
    bufs = (buf0, buf1)
    dstb = (dstb0, dstb1)
    gsem = (gs0, gs1)
    ssem = (ss0, ss1)
    isem = (is0, is1)
    pltpu.async_copy(src_hbm.at[c, s], srcv, gs0)

    def zrow(r, carry):
        for k in range(_H // 16):
            buf0[r, pl.ds(16 * k, 16)] = jnp.zeros((16,), jnp.float32)
        return carry

    lax.fori_loop(0, _CHUNK, zrow, 0)
    for q in range(_RPT // _CHUNK):
        pltpu.sync_copy(buf0, acc_sh.at[pl.ds(s * _RPT + q * _CHUNK, _CHUNK)])
    pltpu.make_async_copy(src_hbm.at[c, s], srcv, gs0).wait()
    plsc.subcore_barrier()

    pltpu.sync_copy(dst_hbm.at[s, 0], dstb0)
    pltpu.sync_copy(dst_hbm.at[s, 1], dstb1)
    pltpu.async_copy(u_hbm.at[srcv.at[0]], bufs[0], gsem[0])

    def step(j0, carry):
        for b in range(2):
            j = 2 * j0 + b

            @pl.when(j + 1 < _NCHUNK)
            def _():
                @pl.when(j >= 1)
                def _():
                    pltpu.make_async_copy(
                        bufs[1 - b], acc_sh.at[dstb[1 - b]], ssem[1 - b]).wait()
                    pltpu.async_copy(dst_hbm.at[s, j + 1], dstb[1 - b],
                                     isem[1 - b])
                pltpu.async_copy(u_hbm.at[srcv.at[j + 1]], bufs[1 - b],
                                 gsem[1 - b])

            pltpu.make_async_copy(u_hbm.at[srcv.at[j]], bufs[b], gsem[b]).wait()

            @pl.when(j >= 2)
            def _():
                pltpu.make_async_copy(dst_hbm.at[s, j], dstb[b], isem[b]).wait()

            pltpu.async_copy(bufs[b], acc_sh.at[dstb[b]], ssem[b], add=True)
        return carry

    lax.fori_loop(0, _NCHUNK // 2, step, 0)
    for b in range(2):
        pltpu.make_async_copy(bufs[b], acc_sh.at[dstb[b]], ssem[b]).wait()
    plsc.subcore_barrier()
    pltpu.sync_copy(acc_sh.at[pl.ds(s * _RPT, _RPT)],
                    out_hbm.at[c, pl.ds(s * _RPT, _RPT)])


@functools.partial(
    pl.kernel,
    out_type=jax.ShapeDtypeStruct((2, _RPAD, _H), jnp.float32),
    mesh=_MESH,
    scratch_types=[
        pltpu.VMEM_SHARED((_RPAD, _H), jnp.float32),
        pltpu.VMEM((_NCHUNK, _CHUNK), jnp.int32),
        pltpu.VMEM((_CHUNK, _H), jnp.float32),
        pltpu.SemaphoreType.DMA,
    ],
)
def _sc_counts(idx_hbm, out_hbm, acc_sh, dstv, buf, sem):
    c = lax.axis_index("c")
    s = lax.axis_index("s")
    pltpu.async_copy(idx_hbm.at[c, s], dstv, sem)

    def zrow(r, carry):
        for k in range(_H // 16):
            buf[r, pl.ds(16 * k, 16)] = jnp.zeros((16,), jnp.float32)
        return carry

    lax.fori_loop(0, _CHUNK, zrow, 0)
    for q in range(_RPT // _CHUNK):
        pltpu.sync_copy(buf, acc_sh.at[pl.ds(s * _RPT + q * _CHUNK, _CHUNK)])

    def orow(r, carry):
        for k in range(_H // 16):
            buf[r, pl.ds(16 * k, 16)] = jnp.ones((16,), jnp.float32)
        return carry

    lax.fori_loop(0, _CHUNK, orow, 0)
    pltpu.make_async_copy(idx_hbm.at[c, s], dstv, sem).wait()
    plsc.subcore_barrier()

    def chunk(g, carry):
        for k in range(8):
            pltpu.async_copy(buf, acc_sh.at[dstv.at[8 * g + k]], sem, add=True)
        for k in range(8):
            pltpu.make_async_copy(buf, acc_sh.at[dstv.at[8 * g + k]], sem).wait()
        return carry

    lax.fori_loop(0, _NCHUNK // 8, chunk, 0)
    plsc.subcore_barrier()
    pltpu.sync_copy(acc_sh.at[pl.ds(s * _RPT, _RPT)],
                    out_hbm.at[c, pl.ds(s * _RPT, _RPT)])


_BM = 2000
_NB = _ND // _BM


def _row_spec(bm=_BM):
    return pl.BlockSpec((1, bm, _H), lambda b, i: (b, i, 0))


def _shared_row_spec(bm=_BM):
    return pl.BlockSpec((1, bm, 1), lambda b, i: (0, i, 0))


_W_SPEC = pl.BlockSpec((_H, _H), lambda b, i: (0, 0))
_V_SPEC = pl.BlockSpec((1, _H), lambda b, i: (0, 0))


def _init_body(detf, errf, detW, detb, errW, errb, w0, hd_o, he_o, u0_o):
    hd = jnp.maximum(detf[...] * detW[...][None] + detb[...][None], 0.0)
    he = jnp.maximum(errf[...] * errW[...][None] + errb[...][None], 0.0)
    hd_o[...] = hd
    he_o[...] = he
    u0_o[...] = jnp.dot(hd[0], w0[...], preferred_element_type=jnp.float32)[None]


_tc_init = pl.pallas_call(
    _init_body,
    grid=(2, _NB),
    in_specs=[
        pl.BlockSpec((1, _BM, 1), lambda b, i: (b, i, 0)),
        _shared_row_spec(),
        _V_SPEC, _V_SPEC, _V_SPEC, _V_SPEC,
        _W_SPEC,
    ],
    out_specs=[_row_spec(), _row_spec(), _row_spec()],
    out_shape=[
        jax.ShapeDtypeStruct((2, _ND, _H), jnp.float32),
        jax.ShapeDtypeStruct((2, _NE, _H), jnp.float32),
        jax.ShapeDtypeStruct((2, _ND, _H), jnp.float32),
    ],
)


def _pre_body(x, wself, b, z_o):
    z_o[...] = (jnp.dot(x[...][0], wself[...], preferred_element_type=jnp.float32)
                + b[...])[None]


_tc_pre = pl.pallas_call(
    _pre_body,
    grid=(2, _NB),
    in_specs=[_row_spec(), _W_SPEC, _V_SPEC],
    out_specs=[_row_spec()],
    out_shape=[jax.ShapeDtypeStruct((2, _ND, _H), jnp.float32)],
)


def _update_body(has_w, *refs):
    if has_w:
        x, acc, z, cnt, wv, g, bl, wnext, out, unext = refs
    else:
        x, acc, z, cnt, g, bl, wnext, out, unext = refs
        wv = None
    xb = x[...][0]
    accb = acc[...][0]
    scale = 1.0 / jnp.maximum(cnt[...][0], 1.0)
    if wv is not None:
        scale = scale * jax.nn.sigmoid(wv[...][0])
    h = z[...][0] + accb * scale
    y = xb + jnp.maximum(h, 0.0)
    m = jnp.mean(y, axis=-1, keepdims=True)
    v = jnp.mean((y - m) ** 2, axis=-1, keepdims=True)
    o = (y - m) * lax.rsqrt(v + 1e-5) * g[...] + bl[...]
    out[...] = o[None]
    unext[...] = jnp.dot(o, wnext[...], preferred_element_type=jnp.float32)[None]


def _make_update(has_w):
    in_specs = [_row_spec(), _row_spec(), _row_spec(), _shared_row_spec()]
    if has_w:
        in_specs.append(_shared_row_spec())
    in_specs += [_V_SPEC, _V_SPEC, _W_SPEC]
    return pl.pallas_call(
        functools.partial(_update_body, has_w),
        grid=(2, _NB),
        in_specs=in_specs,
        out_specs=[_row_spec(), _row_spec()],
        out_shape=[jax.ShapeDtypeStruct((2, _ND, _H), jnp.float32)] * 2,
    )


_tc_update_e = _make_update(True)
_tc_update_d = _make_update(False)


def _update_head_body(x, acc, z, cnt, wv, g, bl, w1, b1, w2, b2,
                      out, ssum, smax):
    i = pl.program_id(1)
    xb = x[...][0]
    accb = acc[...][0]
    scale = jax.nn.sigmoid(wv[...][0]) / jnp.maximum(cnt[...][0], 1.0)
    h = z[...][0] + accb * scale
    y = xb + jnp.maximum(h, 0.0)
    m = jnp.mean(y, axis=-1, keepdims=True)
    v = jnp.mean((y - m) ** 2, axis=-1, keepdims=True)
    o = (y - m) * lax.rsqrt(v + 1e-5) * g[...] + bl[...]
    bs = jnp.sum(o, axis=0, keepdims=True)
    bx = jnp.max(o, axis=0, keepdims=True)

    @pl.when(i == 0)
    def _():
        ssum[...] = bs
        smax[...] = bx

    @pl.when(i > 0)
    def _():
        ssum[...] = ssum[...] + bs
        smax[...] = jnp.maximum(smax[...], bx)

    @pl.when(i == _NB - 1)
    def _():
        mean = ssum[...] * (1.0 / _NE)
        emb = jnp.concatenate([mean, smax[...]], axis=-1)
        hh = jnp.maximum(jnp.dot(emb, w1[...], preferred_element_type=jnp.float32)
                         + b1[...], 0.0)
        out[...] = (jnp.dot(hh, w2[...], preferred_element_type=jnp.float32)
                    + b2[...])[None]


_tc_update_e_head = pl.pallas_call(
    _update_head_body,
    grid=(2, _NB),
    in_specs=[
        _row_spec(), _row_spec(), _row_spec(), _shared_row_spec(),
        _shared_row_spec(), _V_SPEC, _V_SPEC,
        pl.BlockSpec((2 * _H, _H), lambda b, i: (0, 0)),
        _V_SPEC,
        pl.BlockSpec((_H, 1), lambda b, i: (0, 0)),
        pl.BlockSpec((1, 1), lambda b, i: (0, 0)),
    ],
    out_specs=[pl.BlockSpec((1, 1, 1), lambda b, i: (b, 0, 0))],
    out_shape=[jax.ShapeDtypeStruct((2, 1, 1), jnp.float32)],
    scratch_shapes=[
        pltpu.VMEM((1, _H), jnp.float32),
        pltpu.VMEM((1, _H), jnp.float32),
    ],
)


def kernel(det_features, err_features, edge_index_d2e, edge_index_e2d,
           error_weights, observable_mask, det_W, det_b, err_W, err_b,
           W_d2e, W_e_self, b_e, ln_e_g, ln_e_b, W_e2d, W_d_self, b_d,
           ln_d_g, ln_d_b, head_W1, head_b1, head_W2, head_b2):
    d_src, e_dst = edge_index_d2e[0], edge_index_d2e[1]
    e_src, d_dst = edge_index_e2d[0], edge_index_e2d[1]

    npad = _EPAD - _E
    pad_src = jnp.arange(npad, dtype=jnp.int32) % _ND
    pad_dst = _NE + (jnp.arange(npad, dtype=jnp.int32) % (_RPAD - _NE))

    def _pad(a, p):
        return jnp.concatenate([a, p])

    src_d2e = jnp.stack([_pad(d_src, pad_src), _pad(d_src, pad_src) + _ND]
                        ).reshape(2, _TILES, _NCHUNK, _CHUNK)
    dst_d2e = _pad(e_dst, pad_dst).reshape(_TILES, _NCHUNK, _CHUNK)
    src_e2d = jnp.stack([_pad(e_src, pad_src), _pad(e_src, pad_src) + _NE]
                        ).reshape(2, _TILES, _NCHUNK, _CHUNK)
    dst_e2d = _pad(d_dst, pad_dst).reshape(_TILES, _NCHUNK, _CHUNK)
    cnt_idx = jnp.stack([_pad(e_dst, pad_dst), _pad(d_dst, pad_dst)]
                        ).reshape(2, _TILES, _NCHUNK, _CHUNK)

    counts = _sc_counts(cnt_idx)
    cnt_e = counts[0:1, :, 0:1]
    cnt_d = counts[1:2, :, 0:1]
    wv = error_weights.reshape(1, _NE, 1)

    hD, hE, U = _tc_init(
        det_features, err_features.reshape(1, _NE, 1),
        det_W, det_b.reshape(1, _H), err_W, err_b.reshape(1, _H), W_d2e[0])

    for l in range(_L):
        zE = _tc_pre(hE, W_e_self[l], b_e[l].reshape(1, _H))[0]
        accE = _sc_spmm(U.reshape(2 * _ND, _H), src_d2e, dst_d2e)
        eg = ln_e_g[l].reshape(1, _H)
        el = ln_e_b[l].reshape(1, _H)
        if l < _L - 1:
            hE, V = _tc_update_e(hE, accE, zE, cnt_e, wv, eg, el, W_e2d[l])
            zD = _tc_pre(hD, W_d_self[l], b_d[l].reshape(1, _H))[0]
            accD = _sc_spmm(V.reshape(2 * _NE, _H), src_e2d, dst_e2d)
            hD, U = _tc_update_d(hD, accD, zD, cnt_d,
                                 ln_d_g[l].reshape(1, _H),
                                 ln_d_b[l].reshape(1, _H), W_d2e[l + 1])
        else:
            (out,) = _tc_update_e_head(
                hE, accE, zE, cnt_e, wv, eg, el,
                head_W1, head_b1.reshape(1, _H), head_W2,
                head_b2.reshape(1, 1))
    return out.reshape(2, 1)


# speedup vs baseline: 1.0329x; 1.0329x over previous
import functools

import jax
import jax.numpy as jnp
from jax import lax
from jax.experimental import pallas as pl
from jax.experimental.pallas import tpu as pltpu
from jax.experimental.pallas import tpu_sc as plsc

_ND = 10000
_NE = 10000
_E = 160000
_H = 128
_L = 3
_TILES = 16
_CHUNK = 128
_NCHUNK = 80
_EPAD = _TILES * _NCHUNK * _CHUNK
_RPAD = 10240
_RPT = _RPAD // _TILES

_MESH = plsc.VectorSubcoreMesh(core_axis_name="c", subcore_axis_name="s")


@functools.partial(
    pl.kernel,
    out_type=jax.ShapeDtypeStruct((2, _RPAD, _H), jnp.float32),
    mesh=_MESH,
    scratch_types=[
        pltpu.VMEM_SHARED((_RPAD, _H), jnp.float32),
        pltpu.VMEM((_NCHUNK, _CHUNK), jnp.int32),
        pltpu.VMEM((_CHUNK,), jnp.int32),
        pltpu.VMEM((_CHUNK,), jnp.int32),
        pltpu.VMEM((_CHUNK, _H), jnp.float32),
        pltpu.VMEM((_CHUNK, _H), jnp.float32),
        pltpu.SemaphoreType.DMA,
        pltpu.SemaphoreType.DMA,
        pltpu.SemaphoreType.DMA,
        pltpu.SemaphoreType.DMA,
        pltpu.SemaphoreType.DMA,
        pltpu.SemaphoreType.DMA,
    ],
)
def _sc_spmm(u_hbm, src_hbm, dst_hbm, out_hbm, acc_sh, srcv,
             dstb0, dstb1, buf0, buf1, gs0, gs1, ss0, ss1, is0, is1):
    c = lax.axis_index("c")
    s = lax.axis_index("s")
    bufs = (buf0, buf1)
    dstb = (dstb0, dstb1)
    gsem = (gs0, gs1)
    ssem = (ss0, ss1)
    isem = (is0, is1)
    pltpu.async_copy(src_hbm.at[c, s], srcv, gs0)

    def zrow(r, carry):
        for k in range(_H // 16):
            buf0[r, pl.ds(16 * k, 16)] = jnp.zeros((16,), jnp.float32)
        return carry

    lax.fori_loop(0, _CHUNK, zrow, 0)
    for q in range(_RPT // _CHUNK):
        pltpu.async_copy(buf0, acc_sh.at[pl.ds(s * _RPT + q * _CHUNK, _CHUNK)],
                         ss0)
    for q in range(_RPT // _CHUNK):
        pltpu.make_async_copy(buf0, acc_sh.at[pl.ds(s * _RPT + q * _CHUNK,
                                                    _CHUNK)], ss0).wait()
    pltpu.make_async_copy(src_hbm.at[c, s], srcv, gs0).wait()
    plsc.subcore_barrier()

    pltpu.sync_copy(dst_hbm.at[s, 0], dstb0)
    pltpu.sync_copy(dst_hbm.at[s, 1], dstb1)
    pltpu.async_copy(u_hbm.at[srcv.at[0]], bufs[0], gsem[0])

    def step(j0, carry):
        for b in range(2):
            j = 2 * j0 + b

            @pl.when(j + 1 < _NCHUNK)
            def _():
                @pl.when(j >= 1)
                def _():
                    pltpu.make_async_copy(
                        bufs[1 - b], acc_sh.at[dstb[1 - b]], ssem[1 - b]).wait()
                    pltpu.async_copy(dst_hbm.at[s, j + 1], dstb[1 - b],
                                     isem[1 - b])
                pltpu.async_copy(u_hbm.at[srcv.at[j + 1]], bufs[1 - b],
                                 gsem[1 - b])

            pltpu.make_async_copy(u_hbm.at[srcv.at[j]], bufs[b], gsem[b]).wait()

            @pl.when(j >= 2)
            def _():
                pltpu.make_async_copy(dst_hbm.at[s, j], dstb[b], isem[b]).wait()

            pltpu.async_copy(bufs[b], acc_sh.at[dstb[b]], ssem[b], add=True)
        return carry

    lax.fori_loop(0, _NCHUNK // 2, step, 0)
    for b in range(2):
        pltpu.make_async_copy(bufs[b], acc_sh.at[dstb[b]], ssem[b]).wait()
    plsc.subcore_barrier()
    pltpu.sync_copy(acc_sh.at[pl.ds(s * _RPT, _RPT)],
                    out_hbm.at[c, pl.ds(s * _RPT, _RPT)])


@functools.partial(
    pl.kernel,
    out_type=jax.ShapeDtypeStruct((2, _RPAD, _H), jnp.float32),
    mesh=_MESH,
    scratch_types=[
        pltpu.VMEM_SHARED((_RPAD, _H), jnp.float32),
        pltpu.VMEM((_NCHUNK, _CHUNK), jnp.int32),
        pltpu.VMEM((_CHUNK, _H), jnp.float32),
        pltpu.SemaphoreType.DMA,
    ],
)
def _sc_counts(idx_hbm, out_hbm, acc_sh, dstv, buf, sem):
    c = lax.axis_index("c")
    s = lax.axis_index("s")
    pltpu.async_copy(idx_hbm.at[c, s], dstv, sem)

    def zrow(r, carry):
        for k in range(_H // 16):
            buf[r, pl.ds(16 * k, 16)] = jnp.zeros((16,), jnp.float32)
        return carry

    lax.fori_loop(0, _CHUNK, zrow, 0)
    for q in range(_RPT // _CHUNK):
        pltpu.sync_copy(buf, acc_sh.at[pl.ds(s * _RPT + q * _CHUNK, _CHUNK)])

    def orow(r, carry):
        for k in range(_H // 16):
            buf[r, pl.ds(16 * k, 16)] = jnp.ones((16,), jnp.float32)
        return carry

    lax.fori_loop(0, _CHUNK, orow, 0)
    pltpu.make_async_copy(idx_hbm.at[c, s], dstv, sem).wait()
    plsc.subcore_barrier()

    def chunk(g, carry):
        for k in range(8):
            pltpu.async_copy(buf, acc_sh.at[dstv.at[8 * g + k]], sem, add=True)
        for k in range(8):
            pltpu.make_async_copy(buf, acc_sh.at[dstv.at[8 * g + k]], sem).wait()
        return carry

    lax.fori_loop(0, _NCHUNK // 8, chunk, 0)
    plsc.subcore_barrier()
    pltpu.sync_copy(acc_sh.at[pl.ds(s * _RPT, _RPT)],
                    out_hbm.at[c, pl.ds(s * _RPT, _RPT)])


_BM = 2000
_NB = _ND // _BM


def _row_spec(bm=_BM):
    return pl.BlockSpec((1, bm, _H), lambda b, i: (b, i, 0))


def _shared_row_spec(bm=_BM):
    return pl.BlockSpec((1, bm, 1), lambda b, i: (0, i, 0))


_W_SPEC = pl.BlockSpec((_H, _H), lambda b, i: (0, 0))
_V_SPEC = pl.BlockSpec((1, _H), lambda b, i: (0, 0))


def _init_body(detf, errf, detW, detb, errW, errb, w0, hd_o, he_o, u0_o):
    hd = jnp.maximum(detf[...] * detW[...][None] + detb[...][None], 0.0)
    he = jnp.maximum(errf[...] * errW[...][None] + errb[...][None], 0.0)
    hd_o[...] = hd
    he_o[...] = he
    u0_o[...] = jnp.dot(hd[0], w0[...], preferred_element_type=jnp.float32)[None]


_tc_init = pl.pallas_call(
    _init_body,
    grid=(2, _NB),
    in_specs=[
        pl.BlockSpec((1, _BM, 1), lambda b, i: (b, i, 0)),
        _shared_row_spec(),
        _V_SPEC, _V_SPEC, _V_SPEC, _V_SPEC,
        _W_SPEC,
    ],
    out_specs=[_row_spec(), _row_spec(), _row_spec()],
    out_shape=[
        jax.ShapeDtypeStruct((2, _ND, _H), jnp.float32),
        jax.ShapeDtypeStruct((2, _NE, _H), jnp.float32),
        jax.ShapeDtypeStruct((2, _ND, _H), jnp.float32),
    ],
)


def _update_body(has_w, has_next, *refs):
    if has_w:
        (x, acc, cnt, wv, wself, b, g, bl), rest = refs[:8], refs[8:]
    else:
        (x, acc, cnt, wself, b, g, bl), rest = refs[:7], refs[7:]
        wv = None
    if has_next:
        wnext = rest[0]
        out, unext = rest[1], rest[2]
    else:
        out = rest[0]
        unext = None
    xb = x[...][0]
    accb = acc[...][0]
    scale = 1.0 / jnp.maximum(cnt[...][0], 1.0)
    if wv is not None:
        scale = scale * jax.nn.sigmoid(wv[...][0])
    h = jnp.dot(xb, wself[...], preferred_element_type=jnp.float32)
    h = h + accb * scale + b[...]
    y = xb + jnp.maximum(h, 0.0)
    m = jnp.mean(y, axis=-1, keepdims=True)
    v = jnp.mean((y - m) ** 2, axis=-1, keepdims=True)
    o = (y - m) * lax.rsqrt(v + 1e-5) * g[...] + bl[...]
    out[...] = o[None]
    if unext is not None:
        unext[...] = jnp.dot(o, wnext[...], preferred_element_type=jnp.float32)[None]


def _make_update(has_w, has_next):
    in_specs = [_row_spec(), _row_spec(), _shared_row_spec()]
    if has_w:
        in_specs.append(_shared_row_spec())
    in_specs += [_W_SPEC, _V_SPEC, _V_SPEC, _V_SPEC]
    if has_next:
        in_specs.append(_W_SPEC)
        out_specs = [_row_spec(), _row_spec()]
        out_shape = [jax.ShapeDtypeStruct((2, _ND, _H), jnp.float32)] * 2
    else:
        out_specs = [_row_spec()]
        out_shape = [jax.ShapeDtypeStruct((2, _ND, _H), jnp.float32)]
    return pl.pallas_call(
        functools.partial(_update_body, has_w, has_next),
        grid=(2, _NB),
        in_specs=in_specs,
        out_specs=out_specs,
        out_shape=out_shape,
    )


_tc_update_e = _make_update(True, True)
_tc_update_d = _make_update(False, True)


def _update_head_body(x, acc, cnt, wv, wself, b, g, bl, w1, b1, w2, b2,
                      out, ssum, smax):
    i = pl.program_id(1)
    xb = x[...][0]
    accb = acc[...][0]
    scale = jax.nn.sigmoid(wv[...][0]) / jnp.maximum(cnt[...][0], 1.0)
    h = jnp.dot(xb, wself[...], preferred_element_type=jnp.float32)
    h = h + accb * scale + b[...]
    y = xb + jnp.maximum(h, 0.0)
    m = jnp.mean(y, axis=-1, keepdims=True)
    v = jnp.mean((y - m) ** 2, axis=-1, keepdims=True)
    o = (y - m) * lax.rsqrt(v + 1e-5) * g[...] + bl[...]
    bs = jnp.sum(o, axis=0, keepdims=True)
    bx = jnp.max(o, axis=0, keepdims=True)

    @pl.when(i == 0)
    def _():
        ssum[...] = bs
        smax[...] = bx

    @pl.when(i > 0)
    def _():
        ssum[...] = ssum[...] + bs
        smax[...] = jnp.maximum(smax[...], bx)

    @pl.when(i == _NB - 1)
    def _():
        mean = ssum[...] * (1.0 / _NE)
        emb = jnp.concatenate([mean, smax[...]], axis=-1)
        hh = jnp.maximum(jnp.dot(emb, w1[...], preferred_element_type=jnp.float32)
                         + b1[...], 0.0)
        out[...] = (jnp.dot(hh, w2[...], preferred_element_type=jnp.float32)
                    + b2[...])[None]


_tc_update_e_head = pl.pallas_call(
    _update_head_body,
    grid=(2, _NB),
    in_specs=[
        _row_spec(), _row_spec(), _shared_row_spec(), _shared_row_spec(),
        _W_SPEC, _V_SPEC, _V_SPEC, _V_SPEC,
        pl.BlockSpec((2 * _H, _H), lambda b, i: (0, 0)),
        _V_SPEC,
        pl.BlockSpec((_H, 1), lambda b, i: (0, 0)),
        pl.BlockSpec((1, 1), lambda b, i: (0, 0)),
    ],
    out_specs=[pl.BlockSpec((1, 1, 1), lambda b, i: (b, 0, 0))],
    out_shape=[jax.ShapeDtypeStruct((2, 1, 1), jnp.float32)],
    scratch_shapes=[
        pltpu.VMEM((1, _H), jnp.float32),
        pltpu.VMEM((1, _H), jnp.float32),
    ],
)


def kernel(det_features, err_features, edge_index_d2e, edge_index_e2d,
           error_weights, observable_mask, det_W, det_b, err_W, err_b,
           W_d2e, W_e_self, b_e, ln_e_g, ln_e_b, W_e2d, W_d_self, b_d,
           ln_d_g, ln_d_b, head_W1, head_b1, head_W2, head_b2):
    d_src, e_dst = edge_index_d2e[0], edge_index_d2e[1]
    e_src, d_dst = edge_index_e2d[0], edge_index_e2d[1]

    npad = _EPAD - _E
    pad_src = jnp.arange(npad, dtype=jnp.int32) % _ND
    pad_dst = _NE + (jnp.arange(npad, dtype=jnp.int32) % (_RPAD - _NE))

    def _pad(a, p):
        return jnp.concatenate([a, p])

    src_d2e = jnp.stack([_pad(d_src, pad_src), _pad(d_src, pad_src) + _ND]
                        ).reshape(2, _TILES, _NCHUNK, _CHUNK)
    dst_d2e = _pad(e_dst, pad_dst).reshape(_TILES, _NCHUNK, _CHUNK)
    src_e2d = jnp.stack([_pad(e_src, pad_src), _pad(e_src, pad_src) + _NE]
                        ).reshape(2, _TILES, _NCHUNK, _CHUNK)
    dst_e2d = _pad(d_dst, pad_dst).reshape(_TILES, _NCHUNK, _CHUNK)
    cnt_idx = jnp.stack([_pad(e_dst, pad_dst), _pad(d_dst, pad_dst)]
                        ).reshape(2, _TILES, _NCHUNK, _CHUNK)

    counts = _sc_counts(cnt_idx)
    cnt_e = counts[0:1, :, 0:1]
    cnt_d = counts[1:2, :, 0:1]
    wv = error_weights.reshape(1, _NE, 1)

    hD, hE, U = _tc_init(
        det_features, err_features.reshape(1, _NE, 1),
        det_W, det_b.reshape(1, _H), err_W, err_b.reshape(1, _H), W_d2e[0])

    for l in range(_L):
        accE = _sc_spmm(U.reshape(2 * _ND, _H), src_d2e, dst_d2e)
        eb = b_e[l].reshape(1, _H)
        eg = ln_e_g[l].reshape(1, _H)
        el = ln_e_b[l].reshape(1, _H)
        if l < _L - 1:
            hE, V = _tc_update_e(hE, accE, cnt_e, wv, W_e_self[l], eb, eg, el,
                                 W_e2d[l])
            accD = _sc_spmm(V.reshape(2 * _NE, _H), src_e2d, dst_e2d)
            hD, U = _tc_update_d(hD, accD, cnt_d, W_d_self[l],
                                 b_d[l].reshape(1, _H), ln_d_g[l].reshape(1, _H),
                                 ln_d_b[l].reshape(1, _H), W_d2e[l + 1])
        else:
            (out,) = _tc_update_e_head(
                hE, accE, cnt_e, wv, W_e_self[l], eb, eg, el,
                head_W1, head_b1.reshape(1, _H), head_W2,
                head_b2.reshape(1, 1))
    return out.reshape(2, 1)
